# baseline (speedup 1.0000x reference)
import functools

import jax
import jax.numpy as jnp
from jax import lax
from jax.experimental import pallas as pl
from jax.experimental.pallas import tpu as pltpu
from jax.experimental.pallas import tpu_sc as plsc

_NC, _NS, _LANES = 2, 16, 16


def _sc_chain_gather(inputs, prevs, positions, emb, B, T, D):
    mesh = plsc.VectorSubcoreMesh(
        core_axis_name="c", subcore_axis_name="s", num_cores=_NC, num_subcores=_NS
    )

    @functools.partial(
        pl.kernel,
        out_type=(
            jax.ShapeDtypeStruct((T * B, D), jnp.float32),
            jax.ShapeDtypeStruct((B, _LANES), jnp.int32),
        ),
        mesh=mesh,
        compiler_params=pltpu.CompilerParams(needs_layout_passes=False),
        scratch_types=[
            pltpu.VMEM((T,), jnp.int32),
            pltpu.VMEM((T,), jnp.int32),
            pltpu.VMEM((2 * B + _LANES,), jnp.int32),
            pltpu.VMEM((T,), jnp.int32),
            pltpu.VMEM((T,), jnp.int32),
            pltpu.VMEM((T,), jnp.int32),
            pltpu.VMEM((T,), jnp.int32),
            pltpu.VMEM((T // _LANES, _LANES), jnp.int32),
            pltpu.VMEM((T, D), jnp.float32),
            pltpu.VMEM((_LANES,), jnp.int32),
            pltpu.SemaphoreType.DMA,
        ],
    )
    def body(inputs_hbm, prevs_hbm, pos_hbm, emb_hbm, x_hbm, k_hbm,
             prevs_v, inputs_v, pos_v, fa_v, fb_v, tm_v, idx_v, didx_v,
             rows_v, kv_v, sem):
        c = lax.axis_index("c")
        s = lax.axis_index("s")
        b = s * _NC + c
        NCH = T // _LANES

        c1 = pltpu.async_copy(prevs_hbm.at[b], prevs_v, sem)
        c2 = pltpu.async_copy(inputs_hbm.at[b], inputs_v, sem)
        c3 = pltpu.async_copy(pos_hbm, pos_v.at[pl.ds(0, 2 * B)], sem)
        c1.wait()
        c2.wait()
        c3.wait()

        p = pos_v[pl.ds(2 * b + 1, _LANES)][0] - 1

        for ch in range(NCH):
            tm_v[pl.ds(ch * _LANES, _LANES)] = jnp.full((_LANES,), p, jnp.int32)

        fcur, fnxt = prevs_v, fa_v
        nbits = max(1, (T - 1).bit_length())
        for bit in range(nbits):
            for ch in range(NCH):
                m_ids = lax.iota(jnp.int32, _LANES) + ch * _LANES
                take = (m_ids & (1 << bit)) != 0
                v = tm_v[pl.ds(ch * _LANES, _LANES)]
                g = plsc.load_gather(fcur, [v])
                tm_v[pl.ds(ch * _LANES, _LANES)] = jnp.where(take, g, v)
            if bit + 1 < nbits:
                for ch in range(NCH):
                    v = fcur[pl.ds(ch * _LANES, _LANES)]
                    fnxt[pl.ds(ch * _LANES, _LANES)] = plsc.load_gather(fcur, [v])
                fcur, fnxt = fnxt, (fb_v if fnxt is fa_v else fa_v)

        nz = jnp.zeros((_LANES,), jnp.int32)
        for ch in range(NCH):
            v = tm_v[pl.ds(ch * _LANES, _LANES)]
            tokc = plsc.load_gather(inputs_v, [v])
            idx_v[pl.ds((NCH - 1 - ch) * _LANES, _LANES)] = lax.rev(tokc, (0,))
            nz = nz + jnp.where(v > 0, 1, 0)
        k = jnp.sum(nz) + 1

        for ch in range(NCH):
            didx_v[ch, :] = (
                lax.iota(jnp.int32, _LANES) + ch * _LANES
            ) * B + b

        kv_v[...] = jnp.full((_LANES,), k, jnp.int32)
        ck = pltpu.async_copy(kv_v, k_hbm.at[b], sem)

        for ch in range(NCH):
            @pl.when((ch + 1) * _LANES > T - k)
            def _():
                pltpu.async_copy(
                    emb_hbm.at[idx_v.at[pl.ds(ch * _LANES, _LANES)]],
                    rows_v.at[pl.ds(ch * _LANES, _LANES)],
                    sem,
                )

        for ch in range(NCH):
            @pl.when((ch + 1) * _LANES > T - k)
            def _():
                pltpu.make_async_copy(
                    emb_hbm.at[idx_v.at[pl.ds(ch * _LANES, _LANES)]],
                    rows_v.at[pl.ds(ch * _LANES, _LANES)],
                    sem,
                ).wait()

        for ch in range(NCH):
            @pl.when((ch + 1) * _LANES > T - k)
            def _():
                pltpu.async_copy(
                    rows_v.at[pl.ds(ch * _LANES, _LANES)],
                    x_hbm.at[didx_v.at[ch]],
                    sem,
                )

        for ch in range(NCH):
            @pl.when((ch + 1) * _LANES > T - k)
            def _():
                pltpu.make_async_copy(
                    rows_v.at[pl.ds(ch * _LANES, _LANES)],
                    x_hbm.at[didx_v.at[ch]],
                    sem,
                ).wait()

        ck.wait()

    return body(inputs, prevs, positions.reshape(-1), emb)


def _tc_lstm(x2d, kmat, wih_b, whh_b, bias2, wlin_b, blin2, B, T, D, H, L):
    XCH = 128
    _NT = (((1,), (1,)), ((), ()))

    def body(x_ref, k_ref, wih_ref, whh_ref, bih_ref, bhh_ref, wlin_ref,
             blin_ref, out_ref, xg_ref):
        kvec = k_ref[...][:, 0:1]
        kmax = jnp.max(k_ref[...])
        svec = T - kvec

        wih = wih_ref[...].astype(jnp.bfloat16)
        bias = bih_ref[...] + bhh_ref[...]
        spc = XCH // B
        for tc in range(T * B // XCH):
            @pl.when((tc + 1) * spc > T - kmax)
            def _():
                rb = x_ref[pl.ds(tc * XCH, XCH), :].astype(jnp.bfloat16)
                xg = lax.dot_general(
                    rb, wih, _NT, preferred_element_type=jnp.float32
                ) + bias
                xg_ref[pl.ds(tc * spc, spc)] = xg.reshape(spc, B, 4 * H)

        whh = whh_ref[...].astype(jnp.bfloat16)
        G = B // 2

        def cell(j, xg, h, c, sv):
            gates = xg + lax.dot_general(
                h.astype(jnp.bfloat16), whh, _NT,
                preferred_element_type=jnp.float32,
            )
            i_g = jax.nn.sigmoid(gates[:, 0:H])
            f_g = jax.nn.sigmoid(gates[:, H:2 * H])
            g_g = jnp.tanh(gates[:, 2 * H:3 * H])
            o_g = jax.nn.sigmoid(gates[:, 3 * H:4 * H])
            c_new = f_g * c + i_g * g_g
            h_new = o_g * jnp.tanh(c_new)
            valid = j >= sv
            return jnp.where(valid, h_new, 0.0), jnp.where(valid, c_new, 0.0)

        sv0, sv1 = svec[0:G], svec[G:B]

        def step(j, hc):
            ha, ca, hb, cb = hc
            xg = xg_ref[j]
            ha, ca = cell(j, xg[0:G], ha, ca, sv0)
            hb, cb = cell(j, xg[G:B], hb, cb, sv1)
            return ha, ca, hb, cb

        z = jnp.zeros((G, H), jnp.float32)
        ha, _, hb, _ = lax.fori_loop(T - kmax, T, step, (z, z, z, z))
        h = jnp.concatenate([ha, hb], axis=0)
        out_ref[...] = (
            lax.dot_general(
                h.astype(jnp.bfloat16),
                wlin_ref[...].astype(jnp.bfloat16),
                _NT, preferred_element_type=jnp.float32,
            )
            + blin_ref[...]
        )

    return pl.pallas_call(
        body,
        out_shape=jax.ShapeDtypeStruct((B, L), jnp.float32),
        scratch_shapes=[pltpu.VMEM((T, B, 4 * H), jnp.float32)],
    )(x2d, kmat, wih_b, whh_b, bias2[0], bias2[1], wlin_b, blin2)


def kernel(emb, W_ih, W_hh, b_ih, b_hh, W_lin, b_lin, inputs, positions, prevs):
    V, D = emb.shape
    B, T = inputs.shape
    H = W_hh.shape[1]
    L = W_lin.shape[0]

    x2d, kmat = _sc_chain_gather(inputs, prevs, positions, emb, B, T, D)

    bias2 = (b_ih.reshape(1, 4 * H), b_hh.reshape(1, 4 * H))
    blin2 = b_lin.reshape(1, L)

    return _tc_lstm(x2d, kmat, W_ih, W_hh, bias2, W_lin, blin2, B, T, D, H, L)


# speedup vs baseline: 1.0365x; 1.0365x over previous
import functools

import jax
import jax.numpy as jnp
from jax import lax
from jax.experimental import pallas as pl
from jax.experimental.pallas import tpu as pltpu
from jax.experimental.pallas import tpu_sc as plsc

_NC, _NS, _LANES = 2, 16, 16


def _sc_chain_gather(inputs, prevs, positions, emb, B, T, D):
    mesh = plsc.VectorSubcoreMesh(
        core_axis_name="c", subcore_axis_name="s", num_cores=_NC, num_subcores=_NS
    )

    @functools.partial(
        pl.kernel,
        out_type=(
            jax.ShapeDtypeStruct((T * B, D), jnp.float32),
            jax.ShapeDtypeStruct((B, _LANES), jnp.int32),
        ),
        mesh=mesh,
        compiler_params=pltpu.CompilerParams(needs_layout_passes=False),
        scratch_types=[
            pltpu.VMEM((T,), jnp.int32),
            pltpu.VMEM((T,), jnp.int32),
            pltpu.VMEM((2 * B + _LANES,), jnp.int32),
            pltpu.VMEM((T,), jnp.int32),
            pltpu.VMEM((T,), jnp.int32),
            pltpu.VMEM((T,), jnp.int32),
            pltpu.VMEM((T,), jnp.int32),
            pltpu.VMEM((T // _LANES, _LANES), jnp.int32),
            pltpu.VMEM((T, D), jnp.float32),
            pltpu.VMEM((_LANES,), jnp.int32),
            pltpu.SemaphoreType.DMA,
        ],
    )
    def body(inputs_hbm, prevs_hbm, pos_hbm, emb_hbm, x_hbm, k_hbm,
             prevs_v, inputs_v, pos_v, fa_v, fb_v, tm_v, idx_v, didx_v,
             rows_v, kv_v, sem):
        c = lax.axis_index("c")
        s = lax.axis_index("s")
        b = s * _NC + c
        NCH = T // _LANES

        c1 = pltpu.async_copy(prevs_hbm.at[b], prevs_v, sem)
        c2 = pltpu.async_copy(inputs_hbm.at[b], inputs_v, sem)
        c3 = pltpu.async_copy(pos_hbm, pos_v.at[pl.ds(0, 2 * B)], sem)
        c1.wait()
        c2.wait()
        c3.wait()

        p = pos_v[pl.ds(2 * b + 1, _LANES)][0] - 1

        for ch in range(NCH):
            tm_v[pl.ds(ch * _LANES, _LANES)] = jnp.full((_LANES,), p, jnp.int32)

        fcur, fnxt = prevs_v, fa_v
        nbits = max(1, (T - 1).bit_length())
        for bit in range(nbits):
            for ch in range(NCH):
                m_ids = lax.iota(jnp.int32, _LANES) + ch * _LANES
                take = (m_ids & (1 << bit)) != 0
                v = tm_v[pl.ds(ch * _LANES, _LANES)]
                g = plsc.load_gather(fcur, [v])
                tm_v[pl.ds(ch * _LANES, _LANES)] = jnp.where(take, g, v)
            if bit + 1 < nbits:
                for ch in range(NCH):
                    v = fcur[pl.ds(ch * _LANES, _LANES)]
                    fnxt[pl.ds(ch * _LANES, _LANES)] = plsc.load_gather(fcur, [v])
                fcur, fnxt = fnxt, (fb_v if fnxt is fa_v else fa_v)

        nz = jnp.zeros((_LANES,), jnp.int32)
        for ch in range(NCH):
            v = tm_v[pl.ds(ch * _LANES, _LANES)]
            tokc = plsc.load_gather(inputs_v, [v])
            idx_v[pl.ds((NCH - 1 - ch) * _LANES, _LANES)] = lax.rev(tokc, (0,))
            nz = nz + jnp.where(v > 0, 1, 0)
        k = jnp.sum(nz) + 1

        for ch in range(NCH):
            didx_v[ch, :] = (
                lax.iota(jnp.int32, _LANES) + ch * _LANES
            ) * B + b

        kv_v[...] = jnp.full((_LANES,), k, jnp.int32)
        ck = pltpu.async_copy(kv_v, k_hbm.at[b], sem)

        for ch in range(NCH):
            @pl.when((ch + 1) * _LANES > T - k)
            def _():
                pltpu.async_copy(
                    emb_hbm.at[idx_v.at[pl.ds(ch * _LANES, _LANES)]],
                    rows_v.at[pl.ds(ch * _LANES, _LANES)],
                    sem,
                )

        for ch in range(NCH):
            @pl.when((ch + 1) * _LANES > T - k)
            def _():
                pltpu.make_async_copy(
                    emb_hbm.at[idx_v.at[pl.ds(ch * _LANES, _LANES)]],
                    rows_v.at[pl.ds(ch * _LANES, _LANES)],
                    sem,
                ).wait()

        for ch in range(NCH):
            @pl.when((ch + 1) * _LANES > T - k)
            def _():
                pltpu.async_copy(
                    rows_v.at[pl.ds(ch * _LANES, _LANES)],
                    x_hbm.at[didx_v.at[ch]],
                    sem,
                )

        for ch in range(NCH):
            @pl.when((ch + 1) * _LANES > T - k)
            def _():
                pltpu.make_async_copy(
                    rows_v.at[pl.ds(ch * _LANES, _LANES)],
                    x_hbm.at[didx_v.at[ch]],
                    sem,
                ).wait()

        ck.wait()

    return body(inputs, prevs, positions.reshape(-1), emb)


def _tc_lstm(x2d, kmat, wih_b, whh_b, bias2, wlin_b, blin2, B, T, D, H, L):
    XCH = 128

    def body(x_ref, k_ref, wih_ref, whh_ref, bias_ref, wlin_ref, blin_ref,
             out_ref, xg_ref):
        kvec = k_ref[...][:, 0:1]
        kmax = jnp.max(k_ref[...])
        svec = T - kvec

        wih = wih_ref[...]
        bias = bias_ref[...]
        spc = XCH // B
        for tc in range(T * B // XCH):
            @pl.when((tc + 1) * spc > T - kmax)
            def _():
                rb = x_ref[pl.ds(tc * XCH, XCH), :].astype(jnp.bfloat16)
                xg = jnp.dot(rb, wih, preferred_element_type=jnp.float32) + bias
                xg_ref[pl.ds(tc * spc, spc)] = xg.reshape(spc, B, 4 * H)

        whh = whh_ref[...]
        G = B // 2

        def cell(j, xg, h, c, sv):
            gates = xg + jnp.dot(
                h.astype(jnp.bfloat16), whh, preferred_element_type=jnp.float32
            )
            i_g = jax.nn.sigmoid(gates[:, 0:H])
            f_g = jax.nn.sigmoid(gates[:, H:2 * H])
            g_g = jnp.tanh(gates[:, 2 * H:3 * H])
            o_g = jax.nn.sigmoid(gates[:, 3 * H:4 * H])
            c_new = f_g * c + i_g * g_g
            h_new = o_g * jnp.tanh(c_new)
            valid = j >= sv
            return jnp.where(valid, h_new, 0.0), jnp.where(valid, c_new, 0.0)

        sv0, sv1 = svec[0:G], svec[G:B]

        def step(j, hc):
            ha, ca, hb, cb = hc
            xg = xg_ref[j]
            ha, ca = cell(j, xg[0:G], ha, ca, sv0)
            hb, cb = cell(j, xg[G:B], hb, cb, sv1)
            return ha, ca, hb, cb

        z = jnp.zeros((G, H), jnp.float32)
        ha, _, hb, _ = lax.fori_loop(T - kmax, T, step, (z, z, z, z))
        h = jnp.concatenate([ha, hb], axis=0)
        out_ref[...] = (
            jnp.dot(h.astype(jnp.bfloat16), wlin_ref[...],
                    preferred_element_type=jnp.float32)
            + blin_ref[...]
        )

    return pl.pallas_call(
        body,
        out_shape=jax.ShapeDtypeStruct((B, L), jnp.float32),
        scratch_shapes=[pltpu.VMEM((T, B, 4 * H), jnp.float32)],
    )(x2d, kmat, wih_b, whh_b, bias2, wlin_b, blin2)


def kernel(emb, W_ih, W_hh, b_ih, b_hh, W_lin, b_lin, inputs, positions, prevs):
    V, D = emb.shape
    B, T = inputs.shape
    H = W_hh.shape[1]
    L = W_lin.shape[0]

    x2d, kmat = _sc_chain_gather(inputs, prevs, positions, emb, B, T, D)

    wih_b = W_ih.T.astype(jnp.bfloat16)
    whh_b = W_hh.T.astype(jnp.bfloat16)
    bias2 = (b_ih + b_hh).reshape(1, 4 * H)
    wlin_b = W_lin.T.astype(jnp.bfloat16)
    blin2 = b_lin.reshape(1, L)

    return _tc_lstm(x2d, kmat, wih_b, whh_b, bias2, wlin_b, blin2, B, T, D, H, L)


# speedup vs baseline: 1.0565x; 1.0193x over previous
import functools

import jax
import jax.numpy as jnp
from jax import lax
from jax.experimental import pallas as pl
from jax.experimental.pallas import tpu as pltpu
from jax.experimental.pallas import tpu_sc as plsc

_NC, _NS, _LANES = 2, 16, 16


def _sc_chain_gather(inputs, prevs, positions, emb, B, T, D):
    mesh = plsc.VectorSubcoreMesh(
        core_axis_name="c", subcore_axis_name="s", num_cores=_NC, num_subcores=_NS
    )

    @functools.partial(
        pl.kernel,
        out_type=(
            jax.ShapeDtypeStruct((T * B, D), jnp.float32),
            jax.ShapeDtypeStruct((B, _LANES), jnp.int32),
        ),
        mesh=mesh,
        compiler_params=pltpu.CompilerParams(needs_layout_passes=False),
        scratch_types=[
            pltpu.VMEM((T,), jnp.int32),
            pltpu.VMEM((T,), jnp.int32),
            pltpu.VMEM((2 * B + _LANES,), jnp.int32),
            pltpu.VMEM((T,), jnp.int32),
            pltpu.VMEM((T,), jnp.int32),
            pltpu.VMEM((T,), jnp.int32),
            pltpu.VMEM((T,), jnp.int32),
            pltpu.VMEM((T // _LANES, _LANES), jnp.int32),
            pltpu.VMEM((T, D), jnp.float32),
            pltpu.VMEM((_LANES,), jnp.int32),
            pltpu.SemaphoreType.DMA,
        ],
    )
    def body(inputs_hbm, prevs_hbm, pos_hbm, emb_hbm, x_hbm, k_hbm,
             prevs_v, inputs_v, pos_v, fa_v, fb_v, tm_v, idx_v, didx_v,
             rows_v, kv_v, sem):
        c = lax.axis_index("c")
        s = lax.axis_index("s")
        b = s * _NC + c
        NCH = T // _LANES

        c1 = pltpu.async_copy(prevs_hbm.at[b], prevs_v, sem)
        c2 = pltpu.async_copy(inputs_hbm.at[b], inputs_v, sem)
        c3 = pltpu.async_copy(pos_hbm, pos_v.at[pl.ds(0, 2 * B)], sem)
        c1.wait()
        c2.wait()
        c3.wait()

        p = pos_v[pl.ds(2 * b + 1, _LANES)][0] - 1

        for ch in range(NCH):
            tm_v[pl.ds(ch * _LANES, _LANES)] = jnp.full((_LANES,), p, jnp.int32)

        fcur, fnxt = prevs_v, fa_v
        nbits = max(1, (T - 1).bit_length())
        for bit in range(nbits):
            for ch in range(NCH):
                m_ids = lax.iota(jnp.int32, _LANES) + ch * _LANES
                take = (m_ids & (1 << bit)) != 0
                v = tm_v[pl.ds(ch * _LANES, _LANES)]
                g = plsc.load_gather(fcur, [v])
                tm_v[pl.ds(ch * _LANES, _LANES)] = jnp.where(take, g, v)
            if bit + 1 < nbits:
                for ch in range(NCH):
                    v = fcur[pl.ds(ch * _LANES, _LANES)]
                    fnxt[pl.ds(ch * _LANES, _LANES)] = plsc.load_gather(fcur, [v])
                fcur, fnxt = fnxt, (fb_v if fnxt is fa_v else fa_v)

        nz = jnp.zeros((_LANES,), jnp.int32)
        for ch in range(NCH):
            v = tm_v[pl.ds(ch * _LANES, _LANES)]
            tokc = plsc.load_gather(inputs_v, [v])
            idx_v[pl.ds((NCH - 1 - ch) * _LANES, _LANES)] = lax.rev(tokc, (0,))
            nz = nz + jnp.where(v > 0, 1, 0)
        k = jnp.sum(nz) + 1

        for ch in range(NCH):
            didx_v[ch, :] = (
                lax.iota(jnp.int32, _LANES) + ch * _LANES
            ) * B + b

        kv_v[...] = jnp.full((_LANES,), k, jnp.int32)
        ck = pltpu.async_copy(kv_v, k_hbm.at[b], sem)

        for ch in range(NCH):
            @pl.when((ch + 1) * _LANES > T - k)
            def _():
                pltpu.async_copy(
                    emb_hbm.at[idx_v.at[pl.ds(ch * _LANES, _LANES)]],
                    rows_v.at[pl.ds(ch * _LANES, _LANES)],
                    sem,
                )

        for ch in range(NCH):
            @pl.when((ch + 1) * _LANES > T - k)
            def _():
                pltpu.make_async_copy(
                    emb_hbm.at[idx_v.at[pl.ds(ch * _LANES, _LANES)]],
                    rows_v.at[pl.ds(ch * _LANES, _LANES)],
                    sem,
                ).wait()

        for ch in range(NCH):
            @pl.when((ch + 1) * _LANES > T - k)
            def _():
                pltpu.async_copy(
                    rows_v.at[pl.ds(ch * _LANES, _LANES)],
                    x_hbm.at[didx_v.at[ch]],
                    sem,
                )

        for ch in range(NCH):
            @pl.when((ch + 1) * _LANES > T - k)
            def _():
                pltpu.make_async_copy(
                    rows_v.at[pl.ds(ch * _LANES, _LANES)],
                    x_hbm.at[didx_v.at[ch]],
                    sem,
                ).wait()

        ck.wait()

    return body(inputs, prevs, positions.reshape(-1), emb)


def _tc_lstm(x2d, kmat, wih_b, whh_b, bias2, wlin_b, blin2, B, T, D, H, L):
    XCH = 256

    def body(x_ref, k_ref, wih_ref, whh_ref, bias_ref, wlin_ref, blin_ref,
             out_ref, xg_ref):
        kvec = k_ref[...][:, 0:1]
        kmax = jnp.max(k_ref[...])
        svec = T - kvec

        wih = wih_ref[...]
        bias = bias_ref[...]
        spc = XCH // B
        for tc in range(T * B // XCH):
            @pl.when((tc + 1) * spc > T - kmax)
            def _():
                rb = x_ref[pl.ds(tc * XCH, XCH), :].astype(jnp.bfloat16)
                xg = jnp.dot(rb, wih, preferred_element_type=jnp.float32) + bias
                xg_ref[pl.ds(tc * spc, spc)] = xg.reshape(spc, B, 4 * H)

        whh = whh_ref[...]

        def step(j, hc):
            h, c = hc
            gates = xg_ref[j] + jnp.dot(
                h.astype(jnp.bfloat16), whh, preferred_element_type=jnp.float32
            )
            i_g = jax.nn.sigmoid(gates[:, 0:H])
            f_g = jax.nn.sigmoid(gates[:, H:2 * H])
            g_g = jnp.tanh(gates[:, 2 * H:3 * H])
            o_g = jax.nn.sigmoid(gates[:, 3 * H:4 * H])
            c_new = f_g * c + i_g * g_g
            h_new = o_g * jnp.tanh(c_new)
            valid = j >= svec
            return (
                jnp.where(valid, h_new, 0.0),
                jnp.where(valid, c_new, 0.0),
            )

        h0 = jnp.zeros((B, H), jnp.float32)
        h, _ = lax.fori_loop(T - kmax, T, step, (h0, h0))
        out_ref[...] = (
            jnp.dot(h.astype(jnp.bfloat16), wlin_ref[...],
                    preferred_element_type=jnp.float32)
            + blin_ref[...]
        )

    return pl.pallas_call(
        body,
        out_shape=jax.ShapeDtypeStruct((B, L), jnp.float32),
        scratch_shapes=[pltpu.VMEM((T, B, 4 * H), jnp.float32)],
    )(x2d, kmat, wih_b, whh_b, bias2, wlin_b, blin2)


def kernel(emb, W_ih, W_hh, b_ih, b_hh, W_lin, b_lin, inputs, positions, prevs):
    V, D = emb.shape
    B, T = inputs.shape
    H = W_hh.shape[1]
    L = W_lin.shape[0]

    x2d, kmat = _sc_chain_gather(inputs, prevs, positions, emb, B, T, D)

    wih_b = W_ih.T.astype(jnp.bfloat16)
    whh_b = W_hh.T.astype(jnp.bfloat16)
    bias2 = (b_ih + b_hh).reshape(1, 4 * H)
    wlin_b = W_lin.T.astype(jnp.bfloat16)
    blin2 = b_lin.reshape(1, L)

    return _tc_lstm(x2d, kmat, wih_b, whh_b, bias2, wlin_b, blin2, B, T, D, H, L)
